# baseline (speedup 1.0000x reference)
import jax
import jax.numpy as jnp
from jax.experimental import pallas as pl

_BB = 128


def _miner_kernel(a_ref, n_ref, o_ref):
    a = a_ref[...]
    n = n_ref[...]
    BB, N, D = n.shape
    dist = 1.0 - jnp.sum(n * a[:, None, :], axis=2)
    dmin = jnp.min(dist, axis=1, keepdims=True)
    nidx = jax.lax.broadcasted_iota(jnp.int32, (BB, N), 1)
    idx = jnp.min(jnp.where(dist <= dmin, nidx, N), axis=1, keepdims=True)
    idxb = jnp.broadcast_to(idx, (BB, D))
    sel = jax.lax.broadcasted_iota(jnp.int32, (BB, N, D), 1) == idxb[:, None, :]
    o_ref[...] = jnp.sum(jnp.where(sel, n, 0.0), axis=1)


def kernel(anchor, negatives):
    B, N, D = negatives.shape
    return pl.pallas_call(
        _miner_kernel,
        grid=(B // _BB,),
        in_specs=[
            pl.BlockSpec((_BB, D), lambda i: (i, 0)),
            pl.BlockSpec((_BB, N, D), lambda i: (i, 0, 0)),
        ],
        out_specs=pl.BlockSpec((_BB, D), lambda i: (i, 0)),
        out_shape=jax.ShapeDtypeStruct((B, D), negatives.dtype),
    )(anchor, negatives)


# speedup vs baseline: 1.5047x; 1.5047x over previous
import jax
import jax.numpy as jnp
from jax.experimental import pallas as pl

_BB = 128


_NC = 8


def _miner_kernel(a_ref, n_ref, o_ref):
    a = a_ref[...]
    BB, N, D = n_ref.shape
    ab = a[:, None, :]
    cols = []
    for k in range(N // _NC):
        nk = n_ref[:, k * _NC:(k + 1) * _NC, :]
        cols.append(1.0 - jnp.sum(nk * ab, axis=2))
    dist = jnp.concatenate(cols, axis=1)
    dmin = jnp.min(dist, axis=1, keepdims=True)
    nidx = jax.lax.broadcasted_iota(jnp.int32, (BB, N), 1)
    idx = jnp.min(jnp.where(dist <= dmin, nidx, N), axis=1, keepdims=True)
    idxb = jnp.broadcast_to(idx, (BB, D))
    acc = jnp.zeros((BB, _NC, D), dtype=jnp.float32)
    iota = jax.lax.broadcasted_iota(jnp.int32, (BB, _NC, D), 1)
    idx3 = idxb[:, None, :]
    for k in range(N // _NC):
        nk = n_ref[:, k * _NC:(k + 1) * _NC, :]
        sel = (iota + k * _NC) == idx3
        acc = acc + jnp.where(sel, nk, 0.0)
    o_ref[...] = jnp.sum(acc, axis=1)


def kernel(anchor, negatives):
    B, N, D = negatives.shape
    return pl.pallas_call(
        _miner_kernel,
        grid=(B // _BB,),
        in_specs=[
            pl.BlockSpec((_BB, D), lambda i: (i, 0)),
            pl.BlockSpec((_BB, N, D), lambda i: (i, 0, 0)),
        ],
        out_specs=pl.BlockSpec((_BB, D), lambda i: (i, 0)),
        out_shape=jax.ShapeDtypeStruct((B, D), negatives.dtype),
    )(anchor, negatives)


# speedup vs baseline: 1.8847x; 1.2526x over previous
import functools

import jax
import jax.numpy as jnp
from jax import lax
from jax.experimental import pallas as pl
from jax.experimental.pallas import tpu as pltpu
from jax.experimental.pallas import tpu_sc as plsc

_BB = 128
_NC = 8


def _score_kernel(a_ref, n_ref, idx_ref):
    a = a_ref[...]
    BB, N, D = n_ref.shape
    ab = a[:, None, :]
    cols = []
    for k in range(N // _NC):
        nk = n_ref[:, k * _NC:(k + 1) * _NC, :]
        cols.append(1.0 - jnp.sum(nk * ab, axis=2))
    dist = jnp.concatenate(cols, axis=1)
    dmin = jnp.min(dist, axis=1, keepdims=True)
    nidx = jax.lax.broadcasted_iota(jnp.int32, (BB, N), 1)
    idx = jnp.min(jnp.where(dist <= dmin, nidx, N), axis=1, keepdims=True)
    row = (jax.lax.broadcasted_iota(jnp.int32, (BB, 1), 0)
           + pl.program_id(0) * BB)
    idx_ref[...] = row * N + idx


def _tc_scores(anchor, negatives):
    B, N, D = negatives.shape
    return pl.pallas_call(
        _score_kernel,
        grid=(B // _BB,),
        in_specs=[
            pl.BlockSpec((_BB, D), lambda i: (i, 0)),
            pl.BlockSpec((_BB, N, D), lambda i: (i, 0, 0)),
        ],
        out_specs=pl.BlockSpec((_BB, 1), lambda i: (i, 0)),
        out_shape=jax.ShapeDtypeStruct((B, 1), jnp.int32),
    )(anchor, negatives)


def _sc_gather(table, fidx):
    BN, D = table.shape
    B = fidx.shape[0]
    info = plsc.get_sparse_core_info()
    nw = info.num_cores * info.num_subcores
    b_per_w = B // nw
    mesh = plsc.VectorSubcoreMesh(core_axis_name="c", subcore_axis_name="s")

    @functools.partial(
        pl.kernel,
        mesh=mesh,
        out_type=jax.ShapeDtypeStruct((B, D), jnp.float32),
        scratch_types=[
            pltpu.VMEM((b_per_w,), jnp.int32),
            pltpu.VMEM((b_per_w, D), jnp.float32),
            pltpu.SemaphoreType.DMA,
        ],
    )
    def k(table_hbm, idx_hbm, out_hbm, idx_v, rows_v, sem):
        wid = lax.axis_index("s") * info.num_cores + lax.axis_index("c")
        base = wid * b_per_w
        pltpu.sync_copy(idx_hbm.at[pl.ds(base, b_per_w)], idx_v)
        pltpu.async_copy(table_hbm.at[idx_v], rows_v, sem).wait()
        pltpu.sync_copy(rows_v, out_hbm.at[pl.ds(base, b_per_w)])

    return k(table, fidx)


def kernel(anchor, negatives):
    B, N, D = negatives.shape
    fidx = _tc_scores(anchor, negatives).reshape((B,))
    table = negatives.reshape((B * N, D))
    return _sc_gather(table, fidx)


# speedup vs baseline: 1.9737x; 1.0472x over previous
import functools

import jax
import jax.numpy as jnp
from jax import lax
from jax.experimental import pallas as pl
from jax.experimental.pallas import tpu as pltpu
from jax.experimental.pallas import tpu_sc as plsc

_BB = 256
_NC = 8


def _score_kernel(a_ref, n_ref, idx_ref):
    a = a_ref[...]
    BB, N, D = n_ref.shape
    ab = a[:, None, :]
    cols = []
    for k in range(N // _NC):
        nk = n_ref[:, k * _NC:(k + 1) * _NC, :]
        cols.append(1.0 - jnp.sum(nk * ab, axis=2))
    dist = jnp.concatenate(cols, axis=1)
    dmin = jnp.min(dist, axis=1, keepdims=True)
    nidx = jax.lax.broadcasted_iota(jnp.int32, (BB, N), 1)
    idx = jnp.min(jnp.where(dist <= dmin, nidx, N), axis=1, keepdims=True)
    row = (jax.lax.broadcasted_iota(jnp.int32, (BB, 1), 0)
           + pl.program_id(0) * BB)
    idx_ref[...] = row * N + idx


def _tc_scores(anchor, negatives):
    B, N, D = negatives.shape
    return pl.pallas_call(
        _score_kernel,
        grid=(B // _BB,),
        in_specs=[
            pl.BlockSpec((_BB, D), lambda i: (i, 0)),
            pl.BlockSpec((_BB, N, D), lambda i: (i, 0, 0)),
        ],
        out_specs=pl.BlockSpec((_BB, 1), lambda i: (i, 0)),
        out_shape=jax.ShapeDtypeStruct((B, 1), jnp.int32),
    )(anchor, negatives)


def _sc_gather(table, fidx):
    BN, D = table.shape
    B = fidx.shape[0]
    info = plsc.get_sparse_core_info()
    nw = info.num_cores * info.num_subcores
    b_per_w = B // nw
    mesh = plsc.VectorSubcoreMesh(core_axis_name="c", subcore_axis_name="s")

    @functools.partial(
        pl.kernel,
        mesh=mesh,
        out_type=jax.ShapeDtypeStruct((B, D), jnp.float32),
        scratch_types=[
            pltpu.VMEM((b_per_w,), jnp.int32),
            pltpu.VMEM((b_per_w, D), jnp.float32),
            pltpu.SemaphoreType.DMA,
        ],
    )
    def k(table_hbm, idx_hbm, out_hbm, idx_v, rows_v, sem):
        wid = lax.axis_index("s") * info.num_cores + lax.axis_index("c")
        base = wid * b_per_w
        pltpu.sync_copy(idx_hbm.at[pl.ds(base, b_per_w)], idx_v)
        pltpu.async_copy(table_hbm.at[idx_v], rows_v, sem).wait()
        pltpu.sync_copy(rows_v, out_hbm.at[pl.ds(base, b_per_w)])

    return k(table, fidx)


def kernel(anchor, negatives):
    B, N, D = negatives.shape
    fidx = _tc_scores(anchor, negatives).reshape((B,))
    table = negatives.reshape((B * N, D))
    return _sc_gather(table, fidx)
